# baseline (speedup 1.0000x reference)
import functools

import jax
import jax.numpy as jnp
from jax import lax
from jax.experimental import pallas as pl
from jax.experimental.pallas import tpu as pltpu
from jax.experimental.pallas import tpu_sc as plsc

N = 10000
E = 320000
D_IN = 128
D_H = 128
D_A = 64
K = 64
D_CAT = D_H + D_A

NCORES = 2
NSUB = 16
NPAD = 10112
ROWS_PER_SUB = NPAD // NSUB

HALF = D_CAT // 2

CHUNK = 128
EPAD = 327680
ECHUNKS = EPAD // CHUNK
PAD_SRC = 0
PAD_DST = N + 8

DEG_CH_SUB = ECHUNKS // (NCORES * NSUB)
DEG_FIRE = 8

SC_CH_SUB = ECHUNKS // NSUB
SC_BATCH = 40
SC_NBATCH = SC_CH_SUB // SC_BATCH
NBUF = 4

DEG_W = 8

def _sc_deg_body(dstr_hbm, zeros_hbm, ones_hbm, deg0_hbm, deg1_hbm,
                 deg_sp, dstb, ones_v, sem):
  core = lax.axis_index("c")
  sub = lax.axis_index("s")
  row0 = sub * ROWS_PER_SUB

  pltpu.sync_copy(zeros_hbm, deg_sp.at[pl.ds(row0, ROWS_PER_SUB)])
  pltpu.sync_copy(ones_hbm, ones_v)
  chunk0 = (core * NSUB + sub) * DEG_CH_SUB
  pltpu.sync_copy(dstr_hbm.at[pl.ds(chunk0, DEG_CH_SUB)], dstb)
  plsc.subcore_barrier()

  def batch(g, carry):
    for b in range(DEG_FIRE):
      j = g * DEG_FIRE + b
      pltpu.async_copy(ones_v, deg_sp.at[dstb.at[j]], sem, add=True)
    for b in range(DEG_FIRE):
      j = g * DEG_FIRE + b
      pltpu.make_async_copy(ones_v, deg_sp.at[dstb.at[j]], sem).wait()
    return carry

  lax.fori_loop(0, DEG_CH_SUB // DEG_FIRE, batch, 0)
  plsc.subcore_barrier()

  @pl.when(core == 0)
  def _():
    pltpu.sync_copy(deg_sp.at[pl.ds(row0, ROWS_PER_SUB)],
                    deg0_hbm.at[pl.ds(row0, ROWS_PER_SUB), pl.ds(0, DEG_W)])

  @pl.when(core == 1)
  def _():
    pltpu.sync_copy(deg_sp.at[pl.ds(row0, ROWS_PER_SUB)],
                    deg1_hbm.at[pl.ds(row0, ROWS_PER_SUB), pl.ds(0, DEG_W)])


def _sc_scatter_body(y0_hbm, y1_hbm, srcr_hbm, dstr_hbm,
                     acc0_hbm, acc1_hbm,
                     acc_sp, srcb, dstb, *bufs):
  core = lax.axis_index("c")
  sub = lax.axis_index("s")
  row0 = sub * ROWS_PER_SUB
  rows = bufs[:NBUF]
  gsem = bufs[NBUF:2 * NBUF]
  ssem = bufs[2 * NBUF:]

  def run(y_hbm, acc_hbm):
    pltpu.sync_copy(y_hbm.at[pl.ds(row0, ROWS_PER_SUB)],
                    acc_sp.at[pl.ds(row0, ROWS_PER_SUB)])
    plsc.subcore_barrier()
    def gfire(j, b):
      pltpu.async_copy(y_hbm.at[srcb.at[j]], rows[b], gsem[b])

    def gwait(j, b):
      pltpu.make_async_copy(y_hbm.at[srcb.at[j]], rows[b], gsem[b]).wait()

    def sfire(j, b):
      pltpu.async_copy(rows[b], acc_sp.at[dstb.at[j]], ssem[b], add=True)

    def swait(j, b):
      pltpu.make_async_copy(rows[b], acc_sp.at[dstb.at[j]], ssem[b]).wait()

    for k in range(SC_NBATCH):
      chunk0 = sub * SC_CH_SUB + k * SC_BATCH
      pltpu.sync_copy(srcr_hbm.at[pl.ds(chunk0, SC_BATCH)], srcb)
      pltpu.sync_copy(dstr_hbm.at[pl.ds(chunk0, SC_BATCH)], dstb)
      for b in range(NBUF):
        gfire(b, b)

      def group(g, carry):
        for b in range(NBUF):
          j = g * NBUF + b
          gwait(j, b)
          sfire(j, b)
          swait(j, b)
          gfire(j + NBUF, b)
        return carry

      lax.fori_loop(0, SC_BATCH // NBUF - 1, group, 0)
      for b in range(NBUF):
        j = SC_BATCH - NBUF + b
        gwait(j, b)
        sfire(j, b)
        swait(j, b)

    plsc.subcore_barrier()
    pltpu.sync_copy(acc_sp.at[pl.ds(row0, ROWS_PER_SUB)],
                    acc_hbm.at[pl.ds(row0, ROWS_PER_SUB), pl.ds(0, HALF)])

  @pl.when(core == 0)
  def _():
    run(y0_hbm, acc0_hbm)

  @pl.when(core == 1)
  def _():
    run(y1_hbm, acc1_hbm)


@functools.lru_cache(maxsize=None)
def _sc_kernels():
  mesh = plsc.VectorSubcoreMesh(
      core_axis_name="c", subcore_axis_name="s",
      num_cores=NCORES, num_subcores=NSUB,
  )
  params = pltpu.CompilerParams(use_tc_tiling_on_sc=False)
  sc_deg = pl.kernel(
      _sc_deg_body,
      compiler_params=params,
      out_type=(
          jax.ShapeDtypeStruct((NPAD, 128), jnp.float32),
          jax.ShapeDtypeStruct((NPAD, 128), jnp.float32),
      ),
      mesh=mesh,
      scratch_types=[
          pltpu.VMEM_SHARED((NPAD, DEG_W), jnp.float32),
          pltpu.VMEM((DEG_CH_SUB, CHUNK), jnp.int32),
          pltpu.VMEM((CHUNK, DEG_W), jnp.float32),
          pltpu.SemaphoreType.DMA,
      ],
  )
  sc_scatter = pl.kernel(
      _sc_scatter_body,
      compiler_params=params,
      out_type=(
          jax.ShapeDtypeStruct((NPAD, 128), jnp.float32),
          jax.ShapeDtypeStruct((NPAD, 128), jnp.float32),
      ),
      mesh=mesh,
      scratch_types=(
          [pltpu.VMEM_SHARED((NPAD, HALF), jnp.float32)]
          + [pltpu.VMEM((SC_BATCH, CHUNK), jnp.int32)] * 2
          + [pltpu.VMEM((CHUNK, HALF), jnp.float32)] * NBUF
          + [pltpu.SemaphoreType.DMA] * (2 * NBUF)
      ),
  )
  return sc_deg, sc_scatter


def _tc_y_kernel(x_ref, wcat_ref, d0_ref, d1_ref, y0_ref, y1_ref):
  deg = d0_ref[:, 0:1] + d1_ref[:, 0:1] + 1.0
  dinv = lax.rsqrt(deg)
  xw = jnp.dot(x_ref[...], wcat_ref[...], preferred_element_type=jnp.float32)
  y = xw * dinv
  y0_ref[...] = y[:, :HALF]
  y1_ref[...] = y[:, HALF:]


def _tc_final_kernel(acc0_ref, acc1_ref, d0_ref, d1_ref,
                     wlin_ref, be_ref, ba_ref, bl_ref, out_ref):
  deg = d0_ref[:, 0:1] + d1_ref[:, 0:1] + 1.0
  dinv = lax.rsqrt(deg)
  h0 = acc0_ref[:, :HALF] * dinv
  h1 = acc1_ref[:, :HALF] * dinv
  h = jnp.concatenate([h0, h1], axis=1)
  z = jnp.maximum(h[:, :D_H] + be_ref[...], 0.0)
  sp = jnp.maximum(h[:, D_H:] + ba_ref[...], 0.0)
  logits = jnp.dot(sp, wlin_ref[...],
                   preferred_element_type=jnp.float32) + bl_ref[...]
  m = jnp.max(logits, axis=-1, keepdims=True)
  e = jnp.exp(logits - m)
  s = e / jnp.sum(e, axis=-1, keepdims=True)
  part = lax.dot_general(s, z, (((0,), (0,)), ((), ())),
                         preferred_element_type=jnp.float32)

  @pl.when(pl.program_id(0) == 0)
  def _():
    out_ref[...] = jnp.zeros_like(out_ref)

  out_ref[...] += part


TC_Y_BLOCK = 1264
TC_F_BLOCK = 1000


def kernel(x, edge_index, W_embed, b_embed, W_assign, b_assign, W_lin, b_lin):
  pad_col = jnp.tile(
      jnp.array([[PAD_SRC], [PAD_DST]], jnp.int32), (1, EPAD - E))
  ep = jnp.concatenate([edge_index, pad_col], axis=1)
  srcr = ep[0].reshape(ECHUNKS, CHUNK)
  dstr = ep[1].reshape(ECHUNKS, CHUNK)
  W_cat = jnp.concatenate([W_embed, W_assign], axis=1)
  zeros_deg = jnp.zeros((ROWS_PER_SUB, DEG_W), jnp.float32)
  ones_chunk = jnp.ones((CHUNK, DEG_W), jnp.float32)

  sc_deg, sc_scatter = _sc_kernels()
  deg0, deg1 = sc_deg(dstr, zeros_deg, ones_chunk)

  y0, y1 = pl.pallas_call(
      _tc_y_kernel,
      grid=(NPAD // TC_Y_BLOCK,),
      in_specs=[
          pl.BlockSpec((TC_Y_BLOCK, D_IN), lambda i: (i, 0)),
          pl.BlockSpec((D_IN, D_CAT), lambda i: (0, 0)),
          pl.BlockSpec((TC_Y_BLOCK, 128), lambda i: (i, 0)),
          pl.BlockSpec((TC_Y_BLOCK, 128), lambda i: (i, 0)),
      ],
      out_specs=[
          pl.BlockSpec((TC_Y_BLOCK, HALF), lambda i: (i, 0)),
          pl.BlockSpec((TC_Y_BLOCK, HALF), lambda i: (i, 0)),
      ],
      out_shape=(
          jax.ShapeDtypeStruct((NPAD, HALF), jnp.float32),
          jax.ShapeDtypeStruct((NPAD, HALF), jnp.float32),
      ),
  )(x, W_cat, deg0, deg1)

  acc0, acc1 = sc_scatter(y0, y1, srcr, dstr)

  out = pl.pallas_call(
      _tc_final_kernel,
      grid=(N // TC_F_BLOCK,),
      in_specs=[
          pl.BlockSpec((TC_F_BLOCK, 128), lambda i: (i, 0)),
          pl.BlockSpec((TC_F_BLOCK, 128), lambda i: (i, 0)),
          pl.BlockSpec((TC_F_BLOCK, 128), lambda i: (i, 0)),
          pl.BlockSpec((TC_F_BLOCK, 128), lambda i: (i, 0)),
          pl.BlockSpec((D_A, K), lambda i: (0, 0)),
          pl.BlockSpec((1, D_H), lambda i: (0, 0)),
          pl.BlockSpec((1, D_A), lambda i: (0, 0)),
          pl.BlockSpec((1, K), lambda i: (0, 0)),
      ],
      out_specs=pl.BlockSpec((K, D_H), lambda i: (0, 0)),
      out_shape=jax.ShapeDtypeStruct((K, D_H), jnp.float32),
  )(acc0, acc1, deg0, deg1, W_lin,
    b_embed.reshape(1, D_H), b_assign.reshape(1, D_A), b_lin.reshape(1, K))

  return out


# speedup vs baseline: 2.5003x; 2.5003x over previous
import functools

import jax
import jax.numpy as jnp
from jax import lax
from jax.experimental import pallas as pl
from jax.experimental.pallas import tpu as pltpu
from jax.experimental.pallas import tpu_sc as plsc

N = 10000
E = 320000
D_IN = 128
D_H = 128
D_A = 64
K = 64
D_CAT = D_H + D_A

NCORES = 2
NSUB = 16
NPAD = 10112
ROWS_PER_SUB = NPAD // NSUB

HALF = D_CAT // 2

CHUNK = 128
EPAD = 327680
ECHUNKS = EPAD // CHUNK

DEG_CH_SUB = ECHUNKS // (NCORES * NSUB)
DEG_FIRE = 8

SC_CH_SUB = ECHUNKS // NSUB
SC_BATCH = 40
SC_NBATCH = SC_CH_SUB // SC_BATCH
NBUF = 4

DEG_W = 8

def _sc_deg_body(dstr_hbm, zeros_hbm, ones_hbm, deg0_hbm, deg1_hbm,
                 deg_sp, dstb, ones_v, sem):
  core = lax.axis_index("c")
  sub = lax.axis_index("s")
  row0 = sub * ROWS_PER_SUB

  pltpu.sync_copy(zeros_hbm, deg_sp.at[pl.ds(row0, ROWS_PER_SUB)])
  pltpu.sync_copy(ones_hbm, ones_v)
  chunk0 = (core * NSUB + sub) * DEG_CH_SUB
  pltpu.sync_copy(dstr_hbm.at[pl.ds(chunk0, DEG_CH_SUB)], dstb)
  plsc.subcore_barrier()

  def batch(g, carry):
    for b in range(DEG_FIRE):
      j = g * DEG_FIRE + b
      pltpu.async_copy(ones_v, deg_sp.at[dstb.at[j]], sem, add=True)
    for b in range(DEG_FIRE):
      j = g * DEG_FIRE + b
      pltpu.make_async_copy(ones_v, deg_sp.at[dstb.at[j]], sem).wait()
    return carry

  lax.fori_loop(0, DEG_CH_SUB // DEG_FIRE, batch, 0)
  plsc.subcore_barrier()

  @pl.when(core == 0)
  def _():
    pltpu.sync_copy(deg_sp.at[pl.ds(row0, ROWS_PER_SUB)],
                    deg0_hbm.at[pl.ds(row0, ROWS_PER_SUB), pl.ds(0, DEG_W)])

  @pl.when(core == 1)
  def _():
    pltpu.sync_copy(deg_sp.at[pl.ds(row0, ROWS_PER_SUB)],
                    deg1_hbm.at[pl.ds(row0, ROWS_PER_SUB), pl.ds(0, DEG_W)])


def _sc_scatter_body(y0_hbm, y1_hbm, srcr_hbm, dstr_hbm,
                     acc0_hbm, acc1_hbm,
                     acc_sp, srcb, dstb, *bufs):
  core = lax.axis_index("c")
  sub = lax.axis_index("s")
  row0 = sub * ROWS_PER_SUB
  rows = bufs[:NBUF]
  gsem = bufs[NBUF:2 * NBUF]
  ssem = bufs[2 * NBUF:]

  def run(y_hbm, acc_hbm):
    pltpu.sync_copy(y_hbm.at[pl.ds(row0, ROWS_PER_SUB)],
                    acc_sp.at[pl.ds(row0, ROWS_PER_SUB)])
    plsc.subcore_barrier()
    def gfire(j, b):
      pltpu.async_copy(y_hbm.at[srcb.at[j]], rows[b], gsem[b])

    def gwait(j, b):
      pltpu.make_async_copy(y_hbm.at[srcb.at[j]], rows[b], gsem[b]).wait()

    def sfire(j, b):
      pltpu.async_copy(rows[b], acc_sp.at[dstb.at[j]], ssem[b], add=True)

    def swait(j, b):
      pltpu.make_async_copy(rows[b], acc_sp.at[dstb.at[j]], ssem[b]).wait()

    for k in range(SC_NBATCH):
      chunk0 = sub * SC_CH_SUB + k * SC_BATCH
      pltpu.sync_copy(srcr_hbm.at[pl.ds(chunk0, SC_BATCH)], srcb)
      pltpu.sync_copy(dstr_hbm.at[pl.ds(chunk0, SC_BATCH)], dstb)
      for b in range(NBUF):
        gfire(b, b)

      def group(g, carry):
        for b in range(NBUF):
          j = g * NBUF + b
          gwait(j, b)
          sfire(j, b)
          swait(j, b)
          gfire(j + NBUF, b)
        return carry

      lax.fori_loop(0, SC_BATCH // NBUF - 1, group, 0)
      for b in range(NBUF):
        j = SC_BATCH - NBUF + b
        gwait(j, b)
        sfire(j, b)
        swait(j, b)

    plsc.subcore_barrier()
    pltpu.sync_copy(acc_sp.at[pl.ds(row0, ROWS_PER_SUB)],
                    acc_hbm.at[pl.ds(row0, ROWS_PER_SUB), pl.ds(0, HALF)])

  @pl.when(core == 0)
  def _():
    run(y0_hbm, acc0_hbm)

  @pl.when(core == 1)
  def _():
    run(y1_hbm, acc1_hbm)


@functools.lru_cache(maxsize=None)
def _sc_kernels():
  mesh = plsc.VectorSubcoreMesh(
      core_axis_name="c", subcore_axis_name="s",
      num_cores=NCORES, num_subcores=NSUB,
  )
  params = pltpu.CompilerParams(use_tc_tiling_on_sc=False)
  sc_deg = pl.kernel(
      _sc_deg_body,
      compiler_params=params,
      out_type=(
          jax.ShapeDtypeStruct((NPAD, 128), jnp.float32),
          jax.ShapeDtypeStruct((NPAD, 128), jnp.float32),
      ),
      mesh=mesh,
      scratch_types=[
          pltpu.VMEM_SHARED((NPAD, DEG_W), jnp.float32),
          pltpu.VMEM((DEG_CH_SUB, CHUNK), jnp.int32),
          pltpu.VMEM((CHUNK, DEG_W), jnp.float32),
          pltpu.SemaphoreType.DMA,
      ],
  )
  sc_scatter = pl.kernel(
      _sc_scatter_body,
      compiler_params=params,
      out_type=(
          jax.ShapeDtypeStruct((NPAD, 128), jnp.float32),
          jax.ShapeDtypeStruct((NPAD, 128), jnp.float32),
      ),
      mesh=mesh,
      scratch_types=(
          [pltpu.VMEM_SHARED((NPAD, HALF), jnp.float32)]
          + [pltpu.VMEM((SC_BATCH, CHUNK), jnp.int32)] * 2
          + [pltpu.VMEM((CHUNK, HALF), jnp.float32)] * NBUF
          + [pltpu.SemaphoreType.DMA] * (2 * NBUF)
      ),
  )
  return sc_deg, sc_scatter


def _tc_y_kernel(x_ref, wcat_ref, d0_ref, d1_ref, y0_ref, y1_ref):
  deg = d0_ref[:, 0:1] + d1_ref[:, 0:1] + 1.0
  dinv = lax.rsqrt(deg)
  xw = jnp.dot(x_ref[...], wcat_ref[...], preferred_element_type=jnp.float32)
  y = xw * dinv
  y0_ref[...] = y[:, :HALF]
  y1_ref[...] = y[:, HALF:]


def _tc_final_kernel(acc0_ref, acc1_ref, d0_ref, d1_ref,
                     wlin_ref, be_ref, ba_ref, bl_ref, out_ref):
  deg = d0_ref[:, 0:1] + d1_ref[:, 0:1] + 1.0
  dinv = lax.rsqrt(deg)
  h0 = acc0_ref[:, :HALF] * dinv
  h1 = acc1_ref[:, :HALF] * dinv
  h = jnp.concatenate([h0, h1], axis=1)
  z = jnp.maximum(h[:, :D_H] + be_ref[...], 0.0)
  sp = jnp.maximum(h[:, D_H:] + ba_ref[...], 0.0)
  logits = jnp.dot(sp, wlin_ref[...],
                   preferred_element_type=jnp.float32) + bl_ref[...]
  m = jnp.max(logits, axis=-1, keepdims=True)
  e = jnp.exp(logits - m)
  s = e / jnp.sum(e, axis=-1, keepdims=True)
  part = lax.dot_general(s, z, (((0,), (0,)), ((), ())),
                         preferred_element_type=jnp.float32)

  @pl.when(pl.program_id(0) == 0)
  def _():
    out_ref[...] = jnp.zeros_like(out_ref)

  out_ref[...] += part


TC_Y_BLOCK = 1264
TC_F_BLOCK = 1000


def kernel(x, edge_index, W_embed, b_embed, W_assign, b_assign, W_lin, b_lin):
  pad_idx = N + (jnp.arange(EPAD - E, dtype=jnp.int32) % (NPAD - N))
  ep = jnp.concatenate([edge_index, jnp.stack([pad_idx, pad_idx])], axis=1)
  srcr = ep[0].reshape(ECHUNKS, CHUNK)
  dstr = ep[1].reshape(ECHUNKS, CHUNK)
  W_cat = jnp.concatenate([W_embed, W_assign], axis=1)
  zeros_deg = jnp.zeros((ROWS_PER_SUB, DEG_W), jnp.float32)
  ones_chunk = jnp.ones((CHUNK, DEG_W), jnp.float32)

  sc_deg, sc_scatter = _sc_kernels()
  deg0, deg1 = sc_deg(dstr, zeros_deg, ones_chunk)

  y0, y1 = pl.pallas_call(
      _tc_y_kernel,
      grid=(NPAD // TC_Y_BLOCK,),
      in_specs=[
          pl.BlockSpec((TC_Y_BLOCK, D_IN), lambda i: (i, 0)),
          pl.BlockSpec((D_IN, D_CAT), lambda i: (0, 0)),
          pl.BlockSpec((TC_Y_BLOCK, 128), lambda i: (i, 0)),
          pl.BlockSpec((TC_Y_BLOCK, 128), lambda i: (i, 0)),
      ],
      out_specs=[
          pl.BlockSpec((TC_Y_BLOCK, HALF), lambda i: (i, 0)),
          pl.BlockSpec((TC_Y_BLOCK, HALF), lambda i: (i, 0)),
      ],
      out_shape=(
          jax.ShapeDtypeStruct((NPAD, HALF), jnp.float32),
          jax.ShapeDtypeStruct((NPAD, HALF), jnp.float32),
      ),
  )(x, W_cat, deg0, deg1)

  acc0, acc1 = sc_scatter(y0, y1, srcr, dstr)

  out = pl.pallas_call(
      _tc_final_kernel,
      grid=(N // TC_F_BLOCK,),
      in_specs=[
          pl.BlockSpec((TC_F_BLOCK, 128), lambda i: (i, 0)),
          pl.BlockSpec((TC_F_BLOCK, 128), lambda i: (i, 0)),
          pl.BlockSpec((TC_F_BLOCK, 128), lambda i: (i, 0)),
          pl.BlockSpec((TC_F_BLOCK, 128), lambda i: (i, 0)),
          pl.BlockSpec((D_A, K), lambda i: (0, 0)),
          pl.BlockSpec((1, D_H), lambda i: (0, 0)),
          pl.BlockSpec((1, D_A), lambda i: (0, 0)),
          pl.BlockSpec((1, K), lambda i: (0, 0)),
      ],
      out_specs=pl.BlockSpec((K, D_H), lambda i: (0, 0)),
      out_shape=jax.ShapeDtypeStruct((K, D_H), jnp.float32),
  )(acc0, acc1, deg0, deg1, W_lin,
    b_embed.reshape(1, D_H), b_assign.reshape(1, D_A), b_lin.reshape(1, K))

  return out


# speedup vs baseline: 2.5233x; 1.0092x over previous
import functools

import jax
import jax.numpy as jnp
from jax import lax
from jax.experimental import pallas as pl
from jax.experimental.pallas import tpu as pltpu
from jax.experimental.pallas import tpu_sc as plsc

N = 10000
E = 320000
D_IN = 128
D_H = 128
D_A = 64
K = 64
D_CAT = D_H + D_A

NCORES = 2
NSUB = 16
NPAD = 10112
ROWS_PER_SUB = NPAD // NSUB

HALF = D_CAT // 2

CHUNK = 128
EPAD = 327680
ECHUNKS = EPAD // CHUNK

DEG_CH_SUB = ECHUNKS // (NCORES * NSUB)
DEG_FIRE = 8

SC_CH_SUB = ECHUNKS // NSUB
SC_BATCH = 40
SC_NBATCH = SC_CH_SUB // SC_BATCH
NBUF = 4

DEG_W = 8

def _sc_deg_body(dstr_hbm, zeros_hbm, ones_hbm, deg0_hbm, deg1_hbm,
                 deg_sp, dstb, ones_v, sem):
  core = lax.axis_index("c")
  sub = lax.axis_index("s")
  row0 = sub * ROWS_PER_SUB

  pltpu.sync_copy(zeros_hbm, deg_sp.at[pl.ds(row0, ROWS_PER_SUB)])
  pltpu.sync_copy(ones_hbm, ones_v)
  chunk0 = (core * NSUB + sub) * DEG_CH_SUB
  pltpu.sync_copy(dstr_hbm.at[pl.ds(chunk0, DEG_CH_SUB)], dstb)
  plsc.subcore_barrier()

  def batch(g, carry):
    for b in range(DEG_FIRE):
      j = g * DEG_FIRE + b
      pltpu.async_copy(ones_v, deg_sp.at[dstb.at[j]], sem, add=True)
    for b in range(DEG_FIRE):
      j = g * DEG_FIRE + b
      pltpu.make_async_copy(ones_v, deg_sp.at[dstb.at[j]], sem).wait()
    return carry

  lax.fori_loop(0, DEG_CH_SUB // DEG_FIRE, batch, 0)
  plsc.subcore_barrier()

  @pl.when(core == 0)
  def _():
    pltpu.sync_copy(deg_sp.at[pl.ds(row0, ROWS_PER_SUB)],
                    deg0_hbm.at[pl.ds(row0, ROWS_PER_SUB), pl.ds(0, DEG_W)])

  @pl.when(core == 1)
  def _():
    pltpu.sync_copy(deg_sp.at[pl.ds(row0, ROWS_PER_SUB)],
                    deg1_hbm.at[pl.ds(row0, ROWS_PER_SUB), pl.ds(0, DEG_W)])


def _sc_scatter_body(y0_hbm, y1_hbm, srcr_hbm, dstr_hbm,
                     acc0_hbm, acc1_hbm,
                     acc_sp, srcb, dstb, *bufs):
  core = lax.axis_index("c")
  sub = lax.axis_index("s")
  row0 = sub * ROWS_PER_SUB
  rows = bufs[:NBUF]
  gsem = bufs[NBUF:2 * NBUF]
  ssem = bufs[2 * NBUF:]

  def run(y_hbm, acc_hbm):
    pltpu.sync_copy(y_hbm.at[pl.ds(row0, ROWS_PER_SUB)],
                    acc_sp.at[pl.ds(row0, ROWS_PER_SUB)])
    plsc.subcore_barrier()
    def gfire(j, b):
      pltpu.async_copy(y_hbm.at[srcb.at[j]], rows[b], gsem[b])

    def gwait(j, b):
      pltpu.make_async_copy(y_hbm.at[srcb.at[j]], rows[b], gsem[b]).wait()

    def sfire(j, b):
      pltpu.async_copy(rows[b], acc_sp.at[dstb.at[j]], ssem[b], add=True)

    def swait(j, b):
      pltpu.make_async_copy(rows[b], acc_sp.at[dstb.at[j]], ssem[b]).wait()

    for k in range(SC_NBATCH):
      chunk0 = sub * SC_CH_SUB + k * SC_BATCH
      pltpu.sync_copy(srcr_hbm.at[pl.ds(chunk0, SC_BATCH)], srcb)
      pltpu.sync_copy(dstr_hbm.at[pl.ds(chunk0, SC_BATCH)], dstb)
      for b in range(NBUF):
        gfire(b, b)

      def group(g, carry):
        for b in range(NBUF):
          j = g * NBUF + b
          gwait(j, b)
          sfire(j, b)
          swait(j, b)
          gfire(j + NBUF, b)
        return carry

      lax.fori_loop(0, SC_BATCH // NBUF - 1, group, 0)
      for b in range(NBUF):
        j = SC_BATCH - NBUF + b
        gwait(j, b)
        sfire(j, b)
        swait(j, b)

    plsc.subcore_barrier()
    pltpu.sync_copy(acc_sp.at[pl.ds(row0, ROWS_PER_SUB)],
                    acc_hbm.at[pl.ds(row0, ROWS_PER_SUB), pl.ds(0, HALF)])

  @pl.when(core == 0)
  def _():
    run(y0_hbm, acc0_hbm)

  @pl.when(core == 1)
  def _():
    run(y1_hbm, acc1_hbm)


@functools.lru_cache(maxsize=None)
def _sc_kernels():
  mesh = plsc.VectorSubcoreMesh(
      core_axis_name="c", subcore_axis_name="s",
      num_cores=NCORES, num_subcores=NSUB,
  )
  params = pltpu.CompilerParams(use_tc_tiling_on_sc=False)
  sc_deg = pl.kernel(
      _sc_deg_body,
      compiler_params=params,
      out_type=(
          jax.ShapeDtypeStruct((NPAD, 128), jnp.float32),
          jax.ShapeDtypeStruct((NPAD, 128), jnp.float32),
      ),
      mesh=mesh,
      scratch_types=[
          pltpu.VMEM_SHARED((NPAD, DEG_W), jnp.float32),
          pltpu.VMEM((DEG_CH_SUB, CHUNK), jnp.int32),
          pltpu.VMEM((CHUNK, DEG_W), jnp.float32),
          pltpu.SemaphoreType.DMA,
      ],
  )
  sc_scatter = pl.kernel(
      _sc_scatter_body,
      compiler_params=params,
      out_type=(
          jax.ShapeDtypeStruct((NPAD, 128), jnp.float32),
          jax.ShapeDtypeStruct((NPAD, 128), jnp.float32),
      ),
      mesh=mesh,
      scratch_types=(
          [pltpu.VMEM_SHARED((NPAD, HALF), jnp.float32)]
          + [pltpu.VMEM((SC_BATCH, CHUNK), jnp.int32)] * 2
          + [pltpu.VMEM((CHUNK, HALF), jnp.float32)] * NBUF
          + [pltpu.SemaphoreType.DMA] * (2 * NBUF)
      ),
  )
  return sc_deg, sc_scatter


def _tc_y_kernel(x_ref, wcat_ref, d0_ref, d1_ref, y0_ref, y1_ref):
  deg = d0_ref[:, 0:1] + d1_ref[:, 0:1] + 1.0
  dinv = lax.rsqrt(deg)
  xw = jnp.dot(x_ref[...], wcat_ref[...], preferred_element_type=jnp.float32)
  y = xw * dinv
  y0_ref[...] = y[:, :HALF]
  y1_ref[...] = y[:, HALF:]


def _tc_final_kernel(acc0_ref, acc1_ref, d0_ref, d1_ref,
                     wlin_ref, be_ref, ba_ref, bl_ref, out_ref):
  deg = d0_ref[:, 0:1] + d1_ref[:, 0:1] + 1.0
  dinv = lax.rsqrt(deg)
  h0 = acc0_ref[:, :HALF] * dinv
  h1 = acc1_ref[:, :HALF] * dinv
  h = jnp.concatenate([h0, h1], axis=1)
  z = jnp.maximum(h[:, :D_H] + be_ref[...], 0.0)
  sp = jnp.maximum(h[:, D_H:] + ba_ref[...], 0.0)
  logits = jnp.dot(sp, wlin_ref[...],
                   preferred_element_type=jnp.float32) + bl_ref[...]
  m = jnp.max(logits, axis=-1, keepdims=True)
  e = jnp.exp(logits - m)
  s = e / jnp.sum(e, axis=-1, keepdims=True)
  part = lax.dot_general(s, z, (((0,), (0,)), ((), ())),
                         preferred_element_type=jnp.float32)

  @pl.when(pl.program_id(0) == 0)
  def _():
    out_ref[...] = jnp.zeros_like(out_ref)

  out_ref[...] += part


TC_Y_BLOCK = 2528
TC_Y_RS = TC_Y_BLOCK * HALF // 128
Y_ROWS128 = NPAD * HALF // 128
TC_F_BLOCK = 1000


def kernel(x, edge_index, W_embed, b_embed, W_assign, b_assign, W_lin, b_lin):
  pad_idx = N + (jnp.arange(EPAD - E, dtype=jnp.int32) % (NPAD - N))
  ep = jnp.concatenate([edge_index, jnp.stack([pad_idx, pad_idx])], axis=1)
  srcr = ep[0].reshape(ECHUNKS, CHUNK)
  dstr = ep[1].reshape(ECHUNKS, CHUNK)
  W_cat = jnp.concatenate([W_embed, W_assign], axis=1)
  zeros_deg = jnp.zeros((ROWS_PER_SUB, DEG_W), jnp.float32)
  ones_chunk = jnp.ones((CHUNK, DEG_W), jnp.float32)

  sc_deg, sc_scatter = _sc_kernels()
  deg0, deg1 = sc_deg(dstr, zeros_deg, ones_chunk)

  y0, y1 = pl.pallas_call(
      _tc_y_kernel,
      grid=(NPAD // TC_Y_BLOCK,),
      in_specs=[
          pl.BlockSpec((TC_Y_BLOCK, D_IN), lambda i: (i, 0)),
          pl.BlockSpec((D_IN, D_CAT), lambda i: (0, 0)),
          pl.BlockSpec((TC_Y_BLOCK, 128), lambda i: (i, 0)),
          pl.BlockSpec((TC_Y_BLOCK, 128), lambda i: (i, 0)),
      ],
      out_specs=[
          pl.BlockSpec((TC_Y_BLOCK, HALF), lambda i: (i, 0)),
          pl.BlockSpec((TC_Y_BLOCK, HALF), lambda i: (i, 0)),
      ],
      out_shape=(
          jax.ShapeDtypeStruct((NPAD, HALF), jnp.float32),
          jax.ShapeDtypeStruct((NPAD, HALF), jnp.float32),
      ),
  )(x, W_cat, deg0, deg1)

  acc0, acc1 = sc_scatter(y0, y1, srcr, dstr)

  out = pl.pallas_call(
      _tc_final_kernel,
      grid=(N // TC_F_BLOCK,),
      in_specs=[
          pl.BlockSpec((TC_F_BLOCK, 128), lambda i: (i, 0)),
          pl.BlockSpec((TC_F_BLOCK, 128), lambda i: (i, 0)),
          pl.BlockSpec((TC_F_BLOCK, 128), lambda i: (i, 0)),
          pl.BlockSpec((TC_F_BLOCK, 128), lambda i: (i, 0)),
          pl.BlockSpec((D_A, K), lambda i: (0, 0)),
          pl.BlockSpec((1, D_H), lambda i: (0, 0)),
          pl.BlockSpec((1, D_A), lambda i: (0, 0)),
          pl.BlockSpec((1, K), lambda i: (0, 0)),
      ],
      out_specs=pl.BlockSpec((K, D_H), lambda i: (0, 0)),
      out_shape=jax.ShapeDtypeStruct((K, D_H), jnp.float32),
  )(acc0, acc1, deg0, deg1, W_lin,
    b_embed.reshape(1, D_H), b_assign.reshape(1, D_A), b_lin.reshape(1, K))

  return out


# speedup vs baseline: 2.5593x; 1.0142x over previous
import functools

import jax
import jax.numpy as jnp
from jax import lax
from jax.experimental import pallas as pl
from jax.experimental.pallas import tpu as pltpu
from jax.experimental.pallas import tpu_sc as plsc

N = 10000
E = 320000
D_IN = 128
D_H = 128
D_A = 64
K = 64
D_CAT = D_H + D_A

NCORES = 2
NSUB = 16
NPAD = 10112
ROWS_PER_SUB = NPAD // NSUB

HALF = D_CAT // 2

CHUNK = 128
EPAD = 327680
ECHUNKS = EPAD // CHUNK

DEG_CH_SUB = ECHUNKS // (NCORES * NSUB)
DEG_FIRE = 8

SC_CH_SUB = ECHUNKS // NSUB
SC_BATCH = 40
SC_NBATCH = SC_CH_SUB // SC_BATCH
NBUF = 4

DEG_W = 8

def _sc_deg_body(dstr_hbm, zeros_hbm, ones_hbm, deg0_hbm, deg1_hbm,
                 deg_sp, dstb, ones_v, sem):
  core = lax.axis_index("c")
  sub = lax.axis_index("s")
  row0 = sub * ROWS_PER_SUB

  pltpu.sync_copy(zeros_hbm, deg_sp.at[pl.ds(row0, ROWS_PER_SUB)])
  pltpu.sync_copy(ones_hbm, ones_v)
  chunk0 = (core * NSUB + sub) * DEG_CH_SUB
  pltpu.sync_copy(dstr_hbm.at[pl.ds(chunk0, DEG_CH_SUB)], dstb)
  plsc.subcore_barrier()

  def batch(g, carry):
    for b in range(DEG_FIRE):
      j = g * DEG_FIRE + b
      pltpu.async_copy(ones_v, deg_sp.at[dstb.at[j]], sem, add=True)
    for b in range(DEG_FIRE):
      j = g * DEG_FIRE + b
      pltpu.make_async_copy(ones_v, deg_sp.at[dstb.at[j]], sem).wait()
    return carry

  lax.fori_loop(0, DEG_CH_SUB // DEG_FIRE, batch, 0)
  plsc.subcore_barrier()

  @pl.when(core == 0)
  def _():
    pltpu.sync_copy(deg_sp.at[pl.ds(row0, ROWS_PER_SUB)],
                    deg0_hbm.at[pl.ds(row0, ROWS_PER_SUB), pl.ds(0, DEG_W)])

  @pl.when(core == 1)
  def _():
    pltpu.sync_copy(deg_sp.at[pl.ds(row0, ROWS_PER_SUB)],
                    deg1_hbm.at[pl.ds(row0, ROWS_PER_SUB), pl.ds(0, DEG_W)])


def _sc_scatter_body(y0_hbm, y1_hbm, srcr_hbm, dstr_hbm,
                     acc0_hbm, acc1_hbm,
                     acc_sp, srcb, dstb, *bufs):
  core = lax.axis_index("c")
  sub = lax.axis_index("s")
  row0 = sub * ROWS_PER_SUB
  rows = bufs[:NBUF]
  gsem = bufs[NBUF:2 * NBUF]
  ssem = bufs[2 * NBUF:]

  def run(y_hbm, acc_hbm):
    pltpu.sync_copy(y_hbm.at[pl.ds(row0, ROWS_PER_SUB)],
                    acc_sp.at[pl.ds(row0, ROWS_PER_SUB)])
    plsc.subcore_barrier()
    def gfire(j, b):
      pltpu.async_copy(y_hbm.at[srcb.at[j]], rows[b], gsem[b])

    def gwait(j, b):
      pltpu.make_async_copy(y_hbm.at[srcb.at[j]], rows[b], gsem[b]).wait()

    def sfire(j, b):
      pltpu.async_copy(rows[b], acc_sp.at[dstb.at[j]], ssem[b], add=True)

    def swait(j, b):
      pltpu.make_async_copy(rows[b], acc_sp.at[dstb.at[j]], ssem[b]).wait()

    for k in range(SC_NBATCH):
      chunk0 = sub * SC_CH_SUB + k * SC_BATCH
      pltpu.sync_copy(srcr_hbm.at[pl.ds(chunk0, SC_BATCH)], srcb)
      pltpu.sync_copy(dstr_hbm.at[pl.ds(chunk0, SC_BATCH)], dstb)
      for b in range(NBUF):
        gfire(b, b)

      def group(g, carry):
        for b in range(NBUF):
          j = g * NBUF + b
          gwait(j, b)
          sfire(j, b)
          swait(j, b)
          gfire(j + NBUF, b)
        return carry

      lax.fori_loop(0, SC_BATCH // NBUF - 1, group, 0)
      for b in range(NBUF):
        j = SC_BATCH - NBUF + b
        gwait(j, b)
        sfire(j, b)
        swait(j, b)

    plsc.subcore_barrier()
    pltpu.sync_copy(acc_sp.at[pl.ds(row0, ROWS_PER_SUB)],
                    acc_hbm.at[pl.ds(row0, ROWS_PER_SUB), pl.ds(0, HALF)])

  @pl.when(core == 0)
  def _():
    run(y0_hbm, acc0_hbm)

  @pl.when(core == 1)
  def _():
    run(y1_hbm, acc1_hbm)


@functools.lru_cache(maxsize=None)
def _sc_kernels():
  mesh = plsc.VectorSubcoreMesh(
      core_axis_name="c", subcore_axis_name="s",
      num_cores=NCORES, num_subcores=NSUB,
  )
  params = pltpu.CompilerParams(use_tc_tiling_on_sc=False)
  sc_deg = pl.kernel(
      _sc_deg_body,
      compiler_params=params,
      out_type=(
          jax.ShapeDtypeStruct((NPAD, 128), jnp.float32),
          jax.ShapeDtypeStruct((NPAD, 128), jnp.float32),
      ),
      mesh=mesh,
      scratch_types=[
          pltpu.VMEM_SHARED((NPAD, DEG_W), jnp.float32),
          pltpu.VMEM((DEG_CH_SUB, CHUNK), jnp.int32),
          pltpu.VMEM((CHUNK, DEG_W), jnp.float32),
          pltpu.SemaphoreType.DMA,
      ],
  )
  sc_scatter = pl.kernel(
      _sc_scatter_body,
      compiler_params=params,
      out_type=(
          jax.ShapeDtypeStruct((NPAD, 128), jnp.float32),
          jax.ShapeDtypeStruct((NPAD, 128), jnp.float32),
      ),
      mesh=mesh,
      scratch_types=(
          [pltpu.VMEM_SHARED((NPAD, HALF), jnp.float32)]
          + [pltpu.VMEM((SC_BATCH, CHUNK), jnp.int32)] * 2
          + [pltpu.VMEM((CHUNK, HALF), jnp.float32)] * NBUF
          + [pltpu.SemaphoreType.DMA] * (2 * NBUF)
      ),
  )
  return sc_deg, sc_scatter


def _tc_y_kernel(x_ref, wcat_ref, d0_ref, d1_ref, y0_ref, y1_ref):
  deg = d0_ref[:, 0:1] + d1_ref[:, 0:1] + 1.0
  dinv = lax.rsqrt(deg)
  xw = jnp.dot(x_ref[...], wcat_ref[...], preferred_element_type=jnp.float32)
  y = xw * dinv
  y0_ref[...] = y[:, :HALF]
  y1_ref[...] = y[:, HALF:]


def _tc_final_kernel(acc0_ref, acc1_ref, d0_ref, d1_ref,
                     wlin_ref, be_ref, ba_ref, bl_ref, out_ref):
  deg = d0_ref[:, 0:1] + d1_ref[:, 0:1] + 1.0
  dinv = lax.rsqrt(deg)
  h0 = acc0_ref[:, :HALF] * dinv
  h1 = acc1_ref[:, :HALF] * dinv
  h = jnp.concatenate([h0, h1], axis=1)
  z = jnp.maximum(h[:, :D_H] + be_ref[...], 0.0)
  sp = jnp.maximum(h[:, D_H:] + ba_ref[...], 0.0)
  logits = jnp.dot(sp, wlin_ref[...],
                   preferred_element_type=jnp.float32) + bl_ref[...]
  m = jnp.max(logits, axis=-1, keepdims=True)
  e = jnp.exp(logits - m)
  s = e / jnp.sum(e, axis=-1, keepdims=True)
  part = lax.dot_general(s, z, (((0,), (0,)), ((), ())),
                         preferred_element_type=jnp.float32)

  @pl.when(pl.program_id(0) == 0)
  def _():
    out_ref[...] = jnp.zeros_like(out_ref)

  out_ref[...] += part


TC_Y_BLOCK = 2528
TC_Y_RS = TC_Y_BLOCK * HALF // 128
Y_ROWS128 = NPAD * HALF // 128
TC_F_BLOCK = 2000


def kernel(x, edge_index, W_embed, b_embed, W_assign, b_assign, W_lin, b_lin):
  pad_idx = N + (jnp.arange(EPAD - E, dtype=jnp.int32) % (NPAD - N))
  ep = jnp.concatenate([edge_index, jnp.stack([pad_idx, pad_idx])], axis=1)
  srcr = ep[0].reshape(ECHUNKS, CHUNK)
  dstr = ep[1].reshape(ECHUNKS, CHUNK)
  W_cat = jnp.concatenate([W_embed, W_assign], axis=1)
  zeros_deg = jnp.zeros((ROWS_PER_SUB, DEG_W), jnp.float32)
  ones_chunk = jnp.ones((CHUNK, DEG_W), jnp.float32)

  sc_deg, sc_scatter = _sc_kernels()
  deg0, deg1 = sc_deg(dstr, zeros_deg, ones_chunk)

  y0, y1 = pl.pallas_call(
      _tc_y_kernel,
      grid=(NPAD // TC_Y_BLOCK,),
      in_specs=[
          pl.BlockSpec((TC_Y_BLOCK, D_IN), lambda i: (i, 0)),
          pl.BlockSpec((D_IN, D_CAT), lambda i: (0, 0)),
          pl.BlockSpec((TC_Y_BLOCK, 128), lambda i: (i, 0)),
          pl.BlockSpec((TC_Y_BLOCK, 128), lambda i: (i, 0)),
      ],
      out_specs=[
          pl.BlockSpec((TC_Y_BLOCK, HALF), lambda i: (i, 0)),
          pl.BlockSpec((TC_Y_BLOCK, HALF), lambda i: (i, 0)),
      ],
      out_shape=(
          jax.ShapeDtypeStruct((NPAD, HALF), jnp.float32),
          jax.ShapeDtypeStruct((NPAD, HALF), jnp.float32),
      ),
  )(x, W_cat, deg0, deg1)

  acc0, acc1 = sc_scatter(y0, y1, srcr, dstr)

  out = pl.pallas_call(
      _tc_final_kernel,
      grid=(N // TC_F_BLOCK,),
      in_specs=[
          pl.BlockSpec((TC_F_BLOCK, 128), lambda i: (i, 0)),
          pl.BlockSpec((TC_F_BLOCK, 128), lambda i: (i, 0)),
          pl.BlockSpec((TC_F_BLOCK, 128), lambda i: (i, 0)),
          pl.BlockSpec((TC_F_BLOCK, 128), lambda i: (i, 0)),
          pl.BlockSpec((D_A, K), lambda i: (0, 0)),
          pl.BlockSpec((1, D_H), lambda i: (0, 0)),
          pl.BlockSpec((1, D_A), lambda i: (0, 0)),
          pl.BlockSpec((1, K), lambda i: (0, 0)),
      ],
      out_specs=pl.BlockSpec((K, D_H), lambda i: (0, 0)),
      out_shape=jax.ShapeDtypeStruct((K, D_H), jnp.float32),
  )(acc0, acc1, deg0, deg1, W_lin,
    b_embed.reshape(1, D_H), b_assign.reshape(1, D_A), b_lin.reshape(1, K))

  return out


# speedup vs baseline: 2.5617x; 1.0010x over previous
import functools

import jax
import jax.numpy as jnp
from jax import lax
from jax.experimental import pallas as pl
from jax.experimental.pallas import tpu as pltpu
from jax.experimental.pallas import tpu_sc as plsc

N = 10000
E = 320000
D_IN = 128
D_H = 128
D_A = 64
K = 64
D_CAT = D_H + D_A

NCORES = 2
NSUB = 16
NPAD = 10112
ROWS_PER_SUB = NPAD // NSUB

HALF = D_CAT // 2

CHUNK = 128
EPAD = 327680
ECHUNKS = EPAD // CHUNK

DEG_CH_SUB = ECHUNKS // (NCORES * NSUB)
DEG_FIRE = 8

SC_CH_SUB = ECHUNKS // NSUB
SC_BATCH = 40
SC_NBATCH = SC_CH_SUB // SC_BATCH
NBUF = 4

DEG_W = 8

def _sc_deg_body(dstr_hbm, zeros_hbm, ones_hbm, deg0_hbm, deg1_hbm,
                 deg_sp, dstb, ones_v, sem):
  core = lax.axis_index("c")
  sub = lax.axis_index("s")
  row0 = sub * ROWS_PER_SUB

  pltpu.sync_copy(zeros_hbm, deg_sp.at[pl.ds(row0, ROWS_PER_SUB)])
  pltpu.sync_copy(ones_hbm, ones_v)
  chunk0 = (core * NSUB + sub) * DEG_CH_SUB
  pltpu.sync_copy(dstr_hbm.at[pl.ds(chunk0, DEG_CH_SUB)], dstb)
  plsc.subcore_barrier()

  def batch(g, carry):
    for b in range(DEG_FIRE):
      j = g * DEG_FIRE + b
      pltpu.async_copy(ones_v, deg_sp.at[dstb.at[j]], sem, add=True)
    for b in range(DEG_FIRE):
      j = g * DEG_FIRE + b
      pltpu.make_async_copy(ones_v, deg_sp.at[dstb.at[j]], sem).wait()
    return carry

  lax.fori_loop(0, DEG_CH_SUB // DEG_FIRE, batch, 0)
  plsc.subcore_barrier()

  @pl.when(core == 0)
  def _():
    pltpu.sync_copy(deg_sp.at[pl.ds(row0, ROWS_PER_SUB)],
                    deg0_hbm.at[pl.ds(row0, ROWS_PER_SUB), pl.ds(0, DEG_W)])

  @pl.when(core == 1)
  def _():
    pltpu.sync_copy(deg_sp.at[pl.ds(row0, ROWS_PER_SUB)],
                    deg1_hbm.at[pl.ds(row0, ROWS_PER_SUB), pl.ds(0, DEG_W)])


def _sc_scatter_body(y0_hbm, y1_hbm, srcr_hbm, dstr_hbm,
                     acc0_hbm, acc1_hbm,
                     acc_sp, srcb, dstb, *bufs):
  core = lax.axis_index("c")
  sub = lax.axis_index("s")
  row0 = sub * ROWS_PER_SUB
  rows = bufs[:NBUF]
  gsem = bufs[NBUF:2 * NBUF]
  ssem = bufs[2 * NBUF:]

  def run(y_hbm, acc_hbm):
    pltpu.sync_copy(y_hbm.at[pl.ds(row0, ROWS_PER_SUB)],
                    acc_sp.at[pl.ds(row0, ROWS_PER_SUB)])
    plsc.subcore_barrier()
    def gfire(j, b):
      pltpu.async_copy(y_hbm.at[srcb.at[j]], rows[b], gsem[b])

    def gwait(j, b):
      pltpu.make_async_copy(y_hbm.at[srcb.at[j]], rows[b], gsem[b]).wait()

    def sfire(j, b):
      pltpu.async_copy(rows[b], acc_sp.at[dstb.at[j]], ssem[b], add=True)

    def swait(j, b):
      pltpu.make_async_copy(rows[b], acc_sp.at[dstb.at[j]], ssem[b]).wait()

    for k in range(SC_NBATCH):
      chunk0 = sub * SC_CH_SUB + k * SC_BATCH
      pltpu.sync_copy(srcr_hbm.at[pl.ds(chunk0, SC_BATCH)], srcb)
      pltpu.sync_copy(dstr_hbm.at[pl.ds(chunk0, SC_BATCH)], dstb)
      for b in range(NBUF):
        gfire(b, b)

      def group(g, carry):
        for b in range(NBUF):
          j = g * NBUF + b
          gwait(j, b)
          sfire(j, b)
          swait(j, b)
          gfire(j + NBUF, b)
        return carry

      lax.fori_loop(0, SC_BATCH // NBUF - 1, group, 0)
      for b in range(NBUF):
        j = SC_BATCH - NBUF + b
        gwait(j, b)
        sfire(j, b)
        swait(j, b)

    plsc.subcore_barrier()
    pltpu.sync_copy(acc_sp.at[pl.ds(row0, ROWS_PER_SUB)],
                    acc_hbm.at[pl.ds(row0, ROWS_PER_SUB), pl.ds(0, HALF)])

  @pl.when(core == 0)
  def _():
    run(y0_hbm, acc0_hbm)

  @pl.when(core == 1)
  def _():
    run(y1_hbm, acc1_hbm)


@functools.lru_cache(maxsize=None)
def _sc_kernels():
  mesh = plsc.VectorSubcoreMesh(
      core_axis_name="c", subcore_axis_name="s",
      num_cores=NCORES, num_subcores=NSUB,
  )
  params = pltpu.CompilerParams(use_tc_tiling_on_sc=False)
  sc_deg = pl.kernel(
      _sc_deg_body,
      compiler_params=params,
      out_type=(
          jax.ShapeDtypeStruct((NPAD, 128), jnp.float32),
          jax.ShapeDtypeStruct((NPAD, 128), jnp.float32),
      ),
      mesh=mesh,
      scratch_types=[
          pltpu.VMEM_SHARED((NPAD, DEG_W), jnp.float32),
          pltpu.VMEM((DEG_CH_SUB, CHUNK), jnp.int32),
          pltpu.VMEM((CHUNK, DEG_W), jnp.float32),
          pltpu.SemaphoreType.DMA,
      ],
  )
  sc_scatter = pl.kernel(
      _sc_scatter_body,
      compiler_params=params,
      out_type=(
          jax.ShapeDtypeStruct((NPAD, 128), jnp.float32),
          jax.ShapeDtypeStruct((NPAD, 128), jnp.float32),
      ),
      mesh=mesh,
      scratch_types=(
          [pltpu.VMEM_SHARED((NPAD, HALF), jnp.float32)]
          + [pltpu.VMEM((SC_BATCH, CHUNK), jnp.int32)] * 2
          + [pltpu.VMEM((CHUNK, HALF), jnp.float32)] * NBUF
          + [pltpu.SemaphoreType.DMA] * (2 * NBUF)
      ),
  )
  return sc_deg, sc_scatter


def _tc_y_kernel(x_ref, wcat_ref, d0_ref, d1_ref, y0_ref, y1_ref):
  deg = d0_ref[:, 0:1] + d1_ref[:, 0:1] + 1.0
  dinv = lax.rsqrt(deg)
  xw = jnp.dot(x_ref[...], wcat_ref[...], preferred_element_type=jnp.float32)
  y = xw * dinv
  y0_ref[...] = y[:, :HALF]
  y1_ref[...] = y[:, HALF:]


def _tc_final_kernel(acc0_ref, acc1_ref, d0_ref, d1_ref,
                     wlin_ref, be_ref, ba_ref, bl_ref, out_ref):
  deg = d0_ref[:, 0:1] + d1_ref[:, 0:1] + 1.0
  dinv = lax.rsqrt(deg)
  h0 = acc0_ref[:, :HALF] * dinv
  h1 = acc1_ref[:, :HALF] * dinv
  h = jnp.concatenate([h0, h1], axis=1)
  z = jnp.maximum(h[:, :D_H] + be_ref[...], 0.0)
  sp = jnp.maximum(h[:, D_H:] + ba_ref[...], 0.0)
  logits = jnp.dot(sp, wlin_ref[...],
                   preferred_element_type=jnp.float32) + bl_ref[...]
  m = jnp.max(logits, axis=-1, keepdims=True)
  e = jnp.exp(logits - m)
  s = e / jnp.sum(e, axis=-1, keepdims=True)
  part = lax.dot_general(s, z, (((0,), (0,)), ((), ())),
                         preferred_element_type=jnp.float32)

  @pl.when(pl.program_id(0) == 0)
  def _():
    out_ref[...] = jnp.zeros_like(out_ref)

  out_ref[...] += part


TC_Y_BLOCK = 2528
TC_F_BLOCK = 2000


def kernel(x, edge_index, W_embed, b_embed, W_assign, b_assign, W_lin, b_lin):
  pad_idx = N + (jnp.arange(EPAD - E, dtype=jnp.int32) % (NPAD - N))
  ep = jnp.concatenate([edge_index, jnp.stack([pad_idx, pad_idx])], axis=1)
  srcr = ep[0].reshape(ECHUNKS, CHUNK)
  dstr = ep[1].reshape(ECHUNKS, CHUNK)
  W_cat = jnp.concatenate([W_embed, W_assign], axis=1)
  zeros_deg = jnp.zeros((ROWS_PER_SUB, DEG_W), jnp.float32)
  ones_chunk = jnp.ones((CHUNK, DEG_W), jnp.float32)

  sc_deg, sc_scatter = _sc_kernels()
  deg0, deg1 = sc_deg(dstr, zeros_deg, ones_chunk)

  y0, y1 = pl.pallas_call(
      _tc_y_kernel,
      grid=(NPAD // TC_Y_BLOCK,),
      in_specs=[
          pl.BlockSpec((TC_Y_BLOCK, D_IN), lambda i: (i, 0)),
          pl.BlockSpec((D_IN, D_CAT), lambda i: (0, 0)),
          pl.BlockSpec((TC_Y_BLOCK, 128), lambda i: (i, 0)),
          pl.BlockSpec((TC_Y_BLOCK, 128), lambda i: (i, 0)),
      ],
      out_specs=[
          pl.BlockSpec((TC_Y_BLOCK, HALF), lambda i: (i, 0)),
          pl.BlockSpec((TC_Y_BLOCK, HALF), lambda i: (i, 0)),
      ],
      out_shape=(
          jax.ShapeDtypeStruct((NPAD, HALF), jnp.float32),
          jax.ShapeDtypeStruct((NPAD, HALF), jnp.float32),
      ),
  )(x, W_cat, deg0, deg1)

  acc0, acc1 = sc_scatter(y0, y1, srcr, dstr)

  out = pl.pallas_call(
      _tc_final_kernel,
      grid=(N // TC_F_BLOCK,),
      in_specs=[
          pl.BlockSpec((TC_F_BLOCK, 128), lambda i: (i, 0)),
          pl.BlockSpec((TC_F_BLOCK, 128), lambda i: (i, 0)),
          pl.BlockSpec((TC_F_BLOCK, 128), lambda i: (i, 0)),
          pl.BlockSpec((TC_F_BLOCK, 128), lambda i: (i, 0)),
          pl.BlockSpec((D_A, K), lambda i: (0, 0)),
          pl.BlockSpec((1, D_H), lambda i: (0, 0)),
          pl.BlockSpec((1, D_A), lambda i: (0, 0)),
          pl.BlockSpec((1, K), lambda i: (0, 0)),
      ],
      out_specs=pl.BlockSpec((K, D_H), lambda i: (0, 0)),
      out_shape=jax.ShapeDtypeStruct((K, D_H), jnp.float32),
  )(acc0, acc1, deg0, deg1, W_lin,
    b_embed.reshape(1, D_H), b_assign.reshape(1, D_A), b_lin.reshape(1, K))

  return out
